# baseline (speedup 1.0000x reference)
import functools

import jax
import jax.numpy as jnp
import numpy as np
from jax import lax
from jax.experimental import pallas as pl
from jax.experimental.pallas import tpu as pltpu
from jax.experimental.pallas import tpu_sc as plsc

NUM_CORES = 2
NUM_SUBCORES = 16
NUM_WORKERS = NUM_CORES * NUM_SUBCORES
LANES = 16
CHUNK = 128


def _make_sc_kernel(n_rows: int, d_model: int, seq_len: int):
    rows_per_w = n_rows // NUM_WORKERS
    n_chunks = rows_per_w // CHUNK
    assert n_rows == NUM_WORKERS * n_chunks * CHUNK and n_chunks % 2 == 0
    d2 = 2 * d_model
    n_vecs = CHUNK // LANES
    add_vecs = CHUNK * d_model // LANES

    mesh = plsc.VectorSubcoreMesh(
        core_axis_name="c", subcore_axis_name="s",
        num_cores=NUM_CORES, num_subcores=NUM_SUBCORES,
    )

    @functools.partial(
        pl.kernel,
        out_type=jax.ShapeDtypeStruct((n_rows, d_model), jnp.float32),
        mesh=mesh,
        scratch_types=[
            pltpu.VMEM((CHUNK,), jnp.int32),
            pltpu.VMEM((CHUNK,), jnp.int32),
            pltpu.VMEM((CHUNK // 128, 128), jnp.int32),
            pltpu.VMEM((CHUNK // 128, 128), jnp.int32),
            pltpu.VMEM((CHUNK,), jnp.int32),
            pltpu.VMEM((CHUNK,), jnp.int32),
            pltpu.VMEM((CHUNK, 128), jnp.float32),
            pltpu.VMEM((CHUNK, 128), jnp.float32),
            pltpu.VMEM((CHUNK, 64), jnp.float32),
            pltpu.VMEM((CHUNK, 64), jnp.float32),
            pltpu.VMEM((seq_len * d_model,), jnp.float32),
            pltpu.SemaphoreType.DMA,
            pltpu.SemaphoreType.DMA,
            pltpu.SemaphoreType.DMA,
            pltpu.SemaphoreType.DMA,
        ],
        compiler_params=pltpu.CompilerParams(
            use_tc_tiling_on_sc=True, needs_layout_passes=False),
    )
    def sc_kernel(x_hbm, t128_hbm, pos2_hbm, out_hbm,
                  idxc_a, idxc_b, idx2_a, idx2_b, sel_a, sel_b,
                  gath_a, gath_b, outs_a, outs_b, pos_v,
                  gsem_a, gsem_b, osem_a, osem_b):
        wid = lax.axis_index("s") * NUM_CORES + lax.axis_index("c")
        base = wid * rows_per_w
        pltpu.sync_copy(pos2_hbm, pos_v)
        iota16 = lax.iota(jnp.int32, LANES)

        def stage_idx(i, idxc, idx2, par64):
            cbase = pl.multiple_of(base + i * CHUNK, CHUNK)
            pltpu.sync_copy(x_hbm.at[pl.ds(cbase, CHUNK)], idxc)
            for v in range(n_vecs):
                t = idxc[pl.ds(v * LANES, LANES)]
                idx2[v // 8, pl.ds((v % 8) * LANES, LANES)] = (
                    lax.shift_right_logical(t, jnp.int32(1)))
                par64[pl.ds(v * LANES, LANES)] = lax.shift_left(
                    lax.bitwise_and(t, jnp.int32(1)), jnp.int32(6))

        def start_gather(idx2, gath, gsem):
            for k in range(CHUNK // 128):
                pltpu.async_copy(
                    t128_hbm.at[idx2.at[k]], gath.at[pl.ds(k * 128, 128)],
                    gsem)

        def wait_gather(idx2, gath, gsem):
            for k in range(CHUNK // 128):
                pltpu.make_async_copy(
                    t128_hbm.at[idx2.at[k]], gath.at[pl.ds(k * 128, 128)],
                    gsem).wait()

        def select_add(i, gath, par64, outs):
            l0 = lax.rem(base + i * CHUNK, jnp.int32(seq_len))
            iotas = [iota16 + jnp.int32(dv * LANES)
                     for dv in range(d_model // LANES)]

            @plsc.parallel_loop(0, CHUNK, unroll=2)
            def _(r):
                rsplat = lax.broadcast(r, (LANES,))
                par_splat = plsc.load_gather(par64, [rsplat])
                pb = lax.rem(l0 + r, jnp.int32(seq_len)) * d_model
                for dv in range(d_model // LANES):
                    g = plsc.load_gather(
                        gath, [rsplat, par_splat + iotas[dv]])
                    q = pos_v[pl.ds(pb + dv * LANES, LANES)]
                    outs[r, pl.ds(dv * LANES, LANES)] = g + q

        def start_out(i, outs, osem):
            obase = pl.multiple_of(base + i * CHUNK, CHUNK)
            pltpu.async_copy(outs, out_hbm.at[pl.ds(obase, CHUNK)], osem)

        def wait_out(outs, osem):
            pltpu.make_async_copy(
                outs, out_hbm.at[pl.ds(0, CHUNK)], osem).wait()

        bufA = (idxc_a, idx2_a, sel_a, gath_a, outs_a, gsem_a, osem_a)
        bufB = (idxc_b, idx2_b, sel_b, gath_b, outs_b, gsem_b, osem_b)

        stage_idx(0, bufA[0], bufA[1], bufA[2])
        start_gather(bufA[1], bufA[3], bufA[5])
        n_pairs = n_chunks // 2

        def pair_body(i2, carry):
            i = 2 * i2
            (_, idx2A, selA, gathA, outsA, gsemA, osemA) = bufA
            (_, idx2B, selB, gathB, outsB, gsemB, osemB) = bufB

            @pl.when(i2 > 0)
            def _():
                wait_out(outsB, osemB)

            stage_idx(i + 1, bufB[0], idx2B, selB)
            start_gather(idx2B, gathB, gsemB)
            wait_gather(idx2A, gathA, gsemA)
            select_add(i, gathA, selA, outsA)
            start_out(i, outsA, osemA)

            @pl.when(i2 < n_pairs - 1)
            def _():
                wait_out(outsA, osemA)
                stage_idx(i + 2, bufA[0], idx2A, selA)
                start_gather(idx2A, gathA, gsemA)

            wait_gather(idx2B, gathB, gsemB)
            select_add(i + 1, gathB, selB, outsB)
            start_out(i + 1, outsB, osemB)
            return carry

        lax.fori_loop(0, n_pairs, pair_body, 0)
        wait_out(bufA[4], bufA[6])
        wait_out(bufB[4], bufB[6])

    return sc_kernel


def kernel(x, table):
    batch, seq_len = x.shape
    n_vocab, d_model = table.shape
    position = jnp.arange(0, seq_len, dtype=jnp.float32)[:, None]
    div_term = jnp.exp(
        jnp.arange(0, d_model, dtype=jnp.float32) * (-np.log(10000.0) / d_model)
    )
    pos_encoding = jnp.cos(position * div_term)
    pos_flat = pos_encoding.reshape(-1)

    n_rows = batch * seq_len
    t128 = jnp.concatenate([table[0::2], table[1::2]], axis=1)
    sc_kernel = _make_sc_kernel(n_rows, d_model, seq_len)
    out = sc_kernel(x.reshape(n_rows), t128, pos_flat)
    return out.reshape(batch, seq_len, d_model)


# speedup vs baseline: 7.8985x; 7.8985x over previous
import functools

import jax
import jax.numpy as jnp
import numpy as np
from jax import lax
from jax.experimental import pallas as pl
from jax.experimental.pallas import tpu as pltpu
from jax.experimental.pallas import tpu_sc as plsc

NUM_CORES = 2
NUM_SUBCORES = 16
NUM_WORKERS = NUM_CORES * NUM_SUBCORES
LANES = 16
CHUNK = 128


def _make_sc_kernel(n_rows: int, d_model: int, seq_len: int):
    rows_per_w = n_rows // NUM_WORKERS
    n_chunks = rows_per_w // CHUNK
    assert n_rows == NUM_WORKERS * n_chunks * CHUNK and n_chunks % 2 == 0
    d2 = 2 * d_model
    n_vecs = CHUNK // LANES
    add_vecs = CHUNK * d_model // LANES

    mesh = plsc.VectorSubcoreMesh(
        core_axis_name="c", subcore_axis_name="s",
        num_cores=NUM_CORES, num_subcores=NUM_SUBCORES,
    )

    @functools.partial(
        pl.kernel,
        out_type=jax.ShapeDtypeStruct((n_rows, d_model), jnp.float32),
        mesh=mesh,
        scratch_types=[
            pltpu.VMEM((CHUNK,), jnp.int32),
            pltpu.VMEM((CHUNK,), jnp.int32),
            pltpu.VMEM((CHUNK // 128, 128), jnp.int32),
            pltpu.VMEM((CHUNK // 128, 128), jnp.int32),
            pltpu.VMEM((CHUNK,), jnp.int32),
            pltpu.VMEM((CHUNK,), jnp.int32),
            pltpu.VMEM((CHUNK, 128), jnp.float32),
            pltpu.VMEM((CHUNK, 128), jnp.float32),
            pltpu.VMEM((CHUNK, 64), jnp.float32),
            pltpu.VMEM((CHUNK, 64), jnp.float32),
            pltpu.VMEM((seq_len * d_model,), jnp.float32),
            pltpu.SemaphoreType.DMA,
            pltpu.SemaphoreType.DMA,
            pltpu.SemaphoreType.DMA,
            pltpu.SemaphoreType.DMA,
        ],
        compiler_params=pltpu.CompilerParams(
            use_tc_tiling_on_sc=True, needs_layout_passes=False),
    )
    def sc_kernel(x_hbm, t128_hbm, pos2_hbm, out_hbm,
                  idxc_a, idxc_b, idx2_a, idx2_b, sel_a, sel_b,
                  gath_a, gath_b, outs_a, outs_b, pos_v,
                  gsem_a, gsem_b, osem_a, osem_b):
        wid = lax.axis_index("s") * NUM_CORES + lax.axis_index("c")
        base = wid * rows_per_w
        pltpu.sync_copy(pos2_hbm, pos_v)
        iota16 = lax.iota(jnp.int32, LANES)

        def stage_idx(i, idxc, idx2, par64):
            cbase = pl.multiple_of(base + i * CHUNK, CHUNK)
            pltpu.sync_copy(x_hbm.at[pl.ds(cbase, CHUNK)], idxc)
            for v in range(n_vecs):
                t = idxc[pl.ds(v * LANES, LANES)]
                idx2[v // 8, pl.ds((v % 8) * LANES, LANES)] = (
                    lax.shift_right_logical(t, jnp.int32(1)))
                par64[pl.ds(v * LANES, LANES)] = lax.shift_left(
                    lax.bitwise_and(t, jnp.int32(1)), jnp.int32(6))

        def start_gather(idx2, gath, gsem):
            for k in range(CHUNK // 128):
                pltpu.async_copy(
                    t128_hbm.at[idx2.at[k]], gath.at[pl.ds(k * 128, 128)],
                    gsem)

        def wait_gather(idx2, gath, gsem):
            for k in range(CHUNK // 128):
                pltpu.make_async_copy(
                    t128_hbm.at[idx2.at[k]], gath.at[pl.ds(k * 128, 128)],
                    gsem).wait()

        def select_add(i, gath, par64, outs):
            l0 = lax.rem(base + i * CHUNK, jnp.int32(seq_len))
            iotas = [iota16 + jnp.int32(dv * LANES)
                     for dv in range(d_model // LANES)]

            @plsc.parallel_loop(0, CHUNK, unroll=8)
            def _(r):
                rsplat = lax.broadcast(r, (LANES,))
                par_splat = plsc.load_gather(par64, [rsplat])
                pb = lax.rem(l0 + r, jnp.int32(seq_len)) * d_model
                for dv in range(d_model // LANES):
                    g = plsc.load_gather(
                        gath, [rsplat, par_splat + iotas[dv]])
                    q = pos_v[pl.ds(pb + dv * LANES, LANES)]
                    outs[r, pl.ds(dv * LANES, LANES)] = g + q

        def start_out(i, outs, osem):
            obase = pl.multiple_of(base + i * CHUNK, CHUNK)
            pltpu.async_copy(outs, out_hbm.at[pl.ds(obase, CHUNK)], osem)

        def wait_out(outs, osem):
            pltpu.make_async_copy(
                outs, out_hbm.at[pl.ds(0, CHUNK)], osem).wait()

        bufA = (idxc_a, idx2_a, sel_a, gath_a, outs_a, gsem_a, osem_a)
        bufB = (idxc_b, idx2_b, sel_b, gath_b, outs_b, gsem_b, osem_b)

        stage_idx(0, bufA[0], bufA[1], bufA[2])
        start_gather(bufA[1], bufA[3], bufA[5])
        n_pairs = n_chunks // 2

        def pair_body(i2, carry):
            i = 2 * i2
            (_, idx2A, selA, gathA, outsA, gsemA, osemA) = bufA
            (_, idx2B, selB, gathB, outsB, gsemB, osemB) = bufB

            @pl.when(i2 > 0)
            def _():
                wait_out(outsB, osemB)

            stage_idx(i + 1, bufB[0], idx2B, selB)
            start_gather(idx2B, gathB, gsemB)
            wait_gather(idx2A, gathA, gsemA)
            select_add(i, gathA, selA, outsA)
            start_out(i, outsA, osemA)

            @pl.when(i2 < n_pairs - 1)
            def _():
                wait_out(outsA, osemA)
                stage_idx(i + 2, bufA[0], idx2A, selA)
                start_gather(idx2A, gathA, gsemA)

            wait_gather(idx2B, gathB, gsemB)
            select_add(i + 1, gathB, selB, outsB)
            start_out(i + 1, outsB, osemB)
            return carry

        lax.fori_loop(0, n_pairs, pair_body, 0)
        wait_out(bufA[4], bufA[6])
        wait_out(bufB[4], bufB[6])

    return sc_kernel


def kernel(x, table):
    batch, seq_len = x.shape
    n_vocab, d_model = table.shape
    position = jnp.arange(0, seq_len, dtype=jnp.float32)[:, None]
    div_term = jnp.exp(
        jnp.arange(0, d_model, dtype=jnp.float32) * (-np.log(10000.0) / d_model)
    )
    pos_encoding = jnp.cos(position * div_term)
    pos_flat = pos_encoding.reshape(-1)

    n_rows = batch * seq_len
    t128 = table.reshape(n_vocab // 2, 2 * d_model)
    sc_kernel = _make_sc_kernel(n_rows, d_model, seq_len)
    out = sc_kernel(x.reshape(n_rows), t128, pos_flat)
    return out.reshape(batch, seq_len, d_model)


# speedup vs baseline: 8.4245x; 1.0666x over previous
import functools

import jax
import jax.numpy as jnp
import numpy as np
from jax import lax
from jax.experimental import pallas as pl
from jax.experimental.pallas import tpu as pltpu
from jax.experimental.pallas import tpu_sc as plsc

NUM_CORES = 2
NUM_SUBCORES = 16
NUM_WORKERS = NUM_CORES * NUM_SUBCORES
LANES = 16
CHUNK = 128


def _make_sc_kernel(n_rows: int, d_model: int, seq_len: int):
    rows_per_w = n_rows // NUM_WORKERS
    n_chunks = rows_per_w // CHUNK
    assert n_rows == NUM_WORKERS * n_chunks * CHUNK and n_chunks % 2 == 0
    d2 = 2 * d_model
    n_vecs = CHUNK // LANES
    add_vecs = CHUNK * d_model // LANES

    mesh = plsc.VectorSubcoreMesh(
        core_axis_name="c", subcore_axis_name="s",
        num_cores=NUM_CORES, num_subcores=NUM_SUBCORES,
    )

    @functools.partial(
        pl.kernel,
        out_type=jax.ShapeDtypeStruct((n_rows, d_model), jnp.float32),
        mesh=mesh,
        scratch_types=[
            pltpu.VMEM((rows_per_w,), jnp.int32),
            pltpu.VMEM((CHUNK // 128, 128), jnp.int32),
            pltpu.VMEM((CHUNK // 128, 128), jnp.int32),
            pltpu.VMEM((CHUNK,), jnp.int32),
            pltpu.VMEM((CHUNK,), jnp.int32),
            pltpu.VMEM((CHUNK, 128), jnp.float32),
            pltpu.VMEM((CHUNK, 128), jnp.float32),
            pltpu.VMEM((CHUNK, 64), jnp.float32),
            pltpu.VMEM((CHUNK, 64), jnp.float32),
            pltpu.VMEM((seq_len * d_model,), jnp.float32),
            pltpu.SemaphoreType.DMA,
            pltpu.SemaphoreType.DMA,
            pltpu.SemaphoreType.DMA,
            pltpu.SemaphoreType.DMA,
        ],
        compiler_params=pltpu.CompilerParams(
            use_tc_tiling_on_sc=True, needs_layout_passes=False),
    )
    def sc_kernel(x_hbm, t128_hbm, pos2_hbm, out_hbm,
                  idx_all, idx2_a, idx2_b, sel_a, sel_b,
                  gath_a, gath_b, outs_a, outs_b, pos_v,
                  gsem_a, gsem_b, osem_a, osem_b):
        wid = lax.axis_index("s") * NUM_CORES + lax.axis_index("c")
        base = wid * rows_per_w
        pltpu.sync_copy(pos2_hbm, pos_v)
        pltpu.sync_copy(x_hbm.at[pl.ds(base, rows_per_w)], idx_all)
        iota16 = lax.iota(jnp.int32, LANES)

        def stage_idx(i, idx2, par64):
            coff = pl.multiple_of(i * CHUNK, CHUNK)
            for v in range(n_vecs):
                t = idx_all[pl.ds(coff + v * LANES, LANES)]
                idx2[v // 8, pl.ds((v % 8) * LANES, LANES)] = (
                    lax.shift_right_logical(t, jnp.int32(1)))
                par64[pl.ds(v * LANES, LANES)] = lax.shift_left(
                    lax.bitwise_and(t, jnp.int32(1)), jnp.int32(6))

        def start_gather(idx2, gath, gsem):
            for k in range(CHUNK // 128):
                pltpu.async_copy(
                    t128_hbm.at[idx2.at[k]], gath.at[pl.ds(k * 128, 128)],
                    gsem)

        def wait_gather(idx2, gath, gsem):
            for k in range(CHUNK // 128):
                pltpu.make_async_copy(
                    t128_hbm.at[idx2.at[k]], gath.at[pl.ds(k * 128, 128)],
                    gsem).wait()

        def select_add(i, gath, par64, outs):
            l0 = lax.rem(base + i * CHUNK, jnp.int32(seq_len))
            iotas = [iota16 + jnp.int32(dv * LANES)
                     for dv in range(d_model // LANES)]

            @plsc.parallel_loop(0, CHUNK, unroll=8)
            def _(r):
                rsplat = lax.broadcast(r, (LANES,))
                par_splat = plsc.load_gather(par64, [rsplat])
                pb = lax.rem(l0 + r, jnp.int32(seq_len)) * d_model
                for dv in range(d_model // LANES):
                    g = plsc.load_gather(
                        gath, [rsplat, par_splat + iotas[dv]])
                    q = pos_v[pl.ds(pb + dv * LANES, LANES)]
                    outs[r, pl.ds(dv * LANES, LANES)] = g + q

        def start_out(i, outs, osem):
            obase = pl.multiple_of(base + i * CHUNK, CHUNK)
            pltpu.async_copy(outs, out_hbm.at[pl.ds(obase, CHUNK)], osem)

        def wait_out(outs, osem):
            pltpu.make_async_copy(
                outs, out_hbm.at[pl.ds(0, CHUNK)], osem).wait()

        bufA = (idx2_a, sel_a, gath_a, outs_a, gsem_a, osem_a)
        bufB = (idx2_b, sel_b, gath_b, outs_b, gsem_b, osem_b)

        stage_idx(0, bufA[0], bufA[1])
        start_gather(bufA[0], bufA[2], bufA[4])
        n_pairs = n_chunks // 2

        def pair_body(i2, carry):
            i = 2 * i2
            (idx2A, selA, gathA, outsA, gsemA, osemA) = bufA
            (idx2B, selB, gathB, outsB, gsemB, osemB) = bufB

            @pl.when(i2 > 0)
            def _():
                wait_out(outsB, osemB)

            stage_idx(i + 1, idx2B, selB)
            start_gather(idx2B, gathB, gsemB)
            wait_gather(idx2A, gathA, gsemA)
            select_add(i, gathA, selA, outsA)
            start_out(i, outsA, osemA)

            @pl.when(i2 < n_pairs - 1)
            def _():
                wait_out(outsA, osemA)
                stage_idx(i + 2, idx2A, selA)
                start_gather(idx2A, gathA, gsemA)

            wait_gather(idx2B, gathB, gsemB)
            select_add(i + 1, gathB, selB, outsB)
            start_out(i + 1, outsB, osemB)
            return carry

        lax.fori_loop(0, n_pairs, pair_body, 0)
        wait_out(bufA[3], bufA[5])
        wait_out(bufB[3], bufB[5])

    return sc_kernel


def kernel(x, table):
    batch, seq_len = x.shape
    n_vocab, d_model = table.shape
    position = jnp.arange(0, seq_len, dtype=jnp.float32)[:, None]
    div_term = jnp.exp(
        jnp.arange(0, d_model, dtype=jnp.float32) * (-np.log(10000.0) / d_model)
    )
    pos_encoding = jnp.cos(position * div_term)
    pos_flat = pos_encoding.reshape(-1)

    n_rows = batch * seq_len
    t128 = table.reshape(n_vocab // 2, 2 * d_model)
    sc_kernel = _make_sc_kernel(n_rows, d_model, seq_len)
    out = sc_kernel(x.reshape(n_rows), t128, pos_flat)
    return out.reshape(batch, seq_len, d_model)


# speedup vs baseline: 8.6519x; 1.0270x over previous
import functools

import jax
import jax.numpy as jnp
import numpy as np
from jax import lax
from jax.experimental import pallas as pl
from jax.experimental.pallas import tpu as pltpu
from jax.experimental.pallas import tpu_sc as plsc

NUM_CORES = 2
NUM_SUBCORES = 16
NUM_WORKERS = NUM_CORES * NUM_SUBCORES
LANES = 16
CHUNK = 128


def _make_sc_kernel(n_rows: int, d_model: int, seq_len: int):
    rows_per_w = n_rows // NUM_WORKERS
    n_chunks = rows_per_w // CHUNK
    assert n_rows == NUM_WORKERS * n_chunks * CHUNK and n_chunks % 2 == 0
    d2 = 2 * d_model
    n_vecs = CHUNK // LANES
    add_vecs = CHUNK * d_model // LANES

    mesh = plsc.VectorSubcoreMesh(
        core_axis_name="c", subcore_axis_name="s",
        num_cores=NUM_CORES, num_subcores=NUM_SUBCORES,
    )

    @functools.partial(
        pl.kernel,
        out_type=jax.ShapeDtypeStruct((n_rows, d_model), jnp.float32),
        mesh=mesh,
        scratch_types=[
            pltpu.VMEM((rows_per_w,), jnp.int32),
            pltpu.VMEM((CHUNK // 128, 128), jnp.int32),
            pltpu.VMEM((CHUNK // 128, 128), jnp.int32),
            pltpu.VMEM((CHUNK,), jnp.int32),
            pltpu.VMEM((CHUNK,), jnp.int32),
            pltpu.VMEM((CHUNK, 128), jnp.float32),
            pltpu.VMEM((CHUNK, 128), jnp.float32),
            pltpu.VMEM((CHUNK, 64), jnp.float32),
            pltpu.VMEM((CHUNK, 64), jnp.float32),
            pltpu.VMEM((seq_len * d_model,), jnp.float32),
            pltpu.SemaphoreType.DMA,
            pltpu.SemaphoreType.DMA,
            pltpu.SemaphoreType.DMA,
            pltpu.SemaphoreType.DMA,
        ],
        compiler_params=pltpu.CompilerParams(
            use_tc_tiling_on_sc=True, needs_layout_passes=False),
    )
    def sc_kernel(x_hbm, t128_hbm, pos2_hbm, out_hbm,
                  idx_all, idx2_a, idx2_b, sel_a, sel_b,
                  gath_a, gath_b, outs_a, outs_b, pos_v,
                  gsem_a, gsem_b, osem_a, osem_b):
        wid = lax.axis_index("s") * NUM_CORES + lax.axis_index("c")
        base = wid * rows_per_w
        pltpu.sync_copy(pos2_hbm, pos_v)
        pltpu.sync_copy(x_hbm.at[pl.ds(base, rows_per_w)], idx_all)
        iota16 = lax.iota(jnp.int32, LANES)

        def stage_idx(i, idx2, par64):
            coff = pl.multiple_of(i * CHUNK, CHUNK)
            for v in range(n_vecs):
                t = idx_all[pl.ds(coff + v * LANES, LANES)]
                idx2[v // 8, pl.ds((v % 8) * LANES, LANES)] = (
                    lax.shift_right_logical(t, jnp.int32(1)))
                par64[pl.ds(v * LANES, LANES)] = lax.shift_left(
                    lax.bitwise_and(t, jnp.int32(1)), jnp.int32(6))

        def start_gather(idx2, gath, gsem):
            for k in range(CHUNK // 128):
                pltpu.async_copy(
                    t128_hbm.at[idx2.at[k]], gath.at[pl.ds(k * 128, 128)],
                    gsem)

        def wait_gather(idx2, gath, gsem):
            for k in range(CHUNK // 128):
                pltpu.make_async_copy(
                    t128_hbm.at[idx2.at[k]], gath.at[pl.ds(k * 128, 128)],
                    gsem).wait()

        def select_add(i, gath, par64, outs):
            l0 = lax.rem(base + i * CHUNK, jnp.int32(seq_len))
            iotas = [iota16 + jnp.int32(dv * LANES)
                     for dv in range(d_model // LANES)]

            @plsc.parallel_loop(0, CHUNK, unroll=8)
            def _(r):
                rsplat = lax.broadcast(r, (LANES,))
                par_splat = plsc.load_gather(par64, [rsplat])
                pb = lax.rem(l0 + r, jnp.int32(seq_len)) * d_model
                for dv in range(d_model // LANES):
                    g = plsc.load_gather(
                        gath, [rsplat, par_splat + iotas[dv]])
                    q = pos_v[pl.ds(pb + dv * LANES, LANES)]
                    outs[r, pl.ds(dv * LANES, LANES)] = g + q

        def start_out(i, outs, osem):
            obase = pl.multiple_of(base + i * CHUNK, CHUNK)
            pltpu.async_copy(outs, out_hbm.at[pl.ds(obase, CHUNK)], osem)

        def wait_out(outs, osem):
            pltpu.make_async_copy(
                outs, out_hbm.at[pl.ds(0, CHUNK)], osem).wait()

        bufA = (idx2_a, sel_a, gath_a, outs_a, gsem_a, osem_a)
        bufB = (idx2_b, sel_b, gath_b, outs_b, gsem_b, osem_b)

        stage_idx(0, bufA[0], bufA[1])
        start_gather(bufA[0], bufA[2], bufA[4])
        n_pairs = n_chunks // 2

        def pair_body(i2, carry):
            i = 2 * i2
            (idx2A, selA, gathA, outsA, gsemA, osemA) = bufA
            (idx2B, selB, gathB, outsB, gsemB, osemB) = bufB

            stage_idx(i + 1, idx2B, selB)
            start_gather(idx2B, gathB, gsemB)
            wait_gather(idx2A, gathA, gsemA)

            @pl.when(i2 > 0)
            def _():
                wait_out(outsA, osemA)

            select_add(i, gathA, selA, outsA)
            start_out(i, outsA, osemA)

            @pl.when(i2 < n_pairs - 1)
            def _():
                stage_idx(i + 2, idx2A, selA)
                start_gather(idx2A, gathA, gsemA)

            wait_gather(idx2B, gathB, gsemB)

            @pl.when(i2 > 0)
            def _():
                wait_out(outsB, osemB)

            select_add(i + 1, gathB, selB, outsB)
            start_out(i + 1, outsB, osemB)
            return carry

        lax.fori_loop(0, n_pairs, pair_body, 0)
        wait_out(bufA[3], bufA[5])
        wait_out(bufB[3], bufB[5])

    return sc_kernel


def kernel(x, table):
    batch, seq_len = x.shape
    n_vocab, d_model = table.shape
    position = jnp.arange(0, seq_len, dtype=jnp.float32)[:, None]
    div_term = jnp.exp(
        jnp.arange(0, d_model, dtype=jnp.float32) * (-np.log(10000.0) / d_model)
    )
    pos_encoding = jnp.cos(position * div_term)
    pos_flat = pos_encoding.reshape(-1)

    n_rows = batch * seq_len
    t128 = table.reshape(n_vocab // 2, 2 * d_model)
    sc_kernel = _make_sc_kernel(n_rows, d_model, seq_len)
    out = sc_kernel(x.reshape(n_rows), t128, pos_flat)
    return out.reshape(batch, seq_len, d_model)
